# Initial kernel scaffold; baseline (speedup 1.0000x reference)
import jax
import jax.numpy as jnp
from jax.experimental import pallas as pl


def kernel(x, weight):
    raise NotImplementedError("write your pallas kernel here")


# speedup vs baseline: 4.8086x; 4.8086x over previous
import functools

import jax
import jax.numpy as jnp
from jax import lax
from jax.experimental import pallas as pl
from jax.experimental.pallas import tpu as pltpu
from jax.experimental.pallas import tpu_sc as plsc

D = 32
NC = 2
NS = 16
NW = NC * NS
CHUNK = 1024


@functools.partial(jax.jit, static_argnames=("b_total",))
def _gather_flat(idx_flat, weight, b_total):
    b_per_w = b_total // NW
    n_chunks = b_per_w // CHUNK
    mesh = plsc.VectorSubcoreMesh(core_axis_name="c", subcore_axis_name="s")

    @functools.partial(
        pl.kernel,
        mesh=mesh,
        out_type=jax.ShapeDtypeStruct((b_total, D), jnp.float32),
        scratch_types=[
            pltpu.VMEM((CHUNK,), jnp.int32),
            pltpu.VMEM((CHUNK, D), jnp.float32),
            pltpu.SemaphoreType.DMA,
        ],
        compiler_params=pltpu.CompilerParams(use_tc_tiling_on_sc=False),
    )
    def k(idx_hbm, table_hbm, out_hbm, idx_v, rows_v, sem):
        wid = lax.axis_index("s") * NC + lax.axis_index("c")
        base = wid * b_per_w

        def step(c, carry):
            off = base + c * CHUNK
            pltpu.sync_copy(idx_hbm.at[pl.ds(off, CHUNK)], idx_v)
            pltpu.async_copy(table_hbm.at[idx_v], rows_v, sem).wait()
            pltpu.sync_copy(rows_v, out_hbm.at[pl.ds(off, CHUNK)])
            return carry

        lax.fori_loop(0, n_chunks, step, 0)

    return k(idx_flat, weight)


def kernel(x, weight):
    rows, cols = x.shape
    b_total = rows * cols
    flat = x.reshape(b_total).astype(jnp.int32)
    out = _gather_flat(flat, weight, b_total)
    return out.reshape(rows, cols, D)


# speedup vs baseline: 5.0482x; 1.0498x over previous
import functools

import jax
import jax.numpy as jnp
from jax import lax
from jax.experimental import pallas as pl
from jax.experimental.pallas import tpu as pltpu
from jax.experimental.pallas import tpu_sc as plsc

D = 32
NC = 2
NS = 16
NW = NC * NS
CHUNK = 800
NBUF = 4


@functools.partial(jax.jit, static_argnames=("b_total",))
def _gather_flat(idx_flat, weight, b_total):
    b_per_w = b_total // NW
    n_chunks = b_per_w // CHUNK
    mesh = plsc.VectorSubcoreMesh(core_axis_name="c", subcore_axis_name="s")

    @functools.partial(
        pl.kernel,
        mesh=mesh,
        out_type=jax.ShapeDtypeStruct((b_total, D), jnp.float32),
        scratch_types=[
            pltpu.VMEM((NBUF, CHUNK), jnp.int32),
            pltpu.VMEM((NBUF, CHUNK, D), jnp.float32),
            [pltpu.SemaphoreType.DMA] * NBUF,
            [pltpu.SemaphoreType.DMA] * NBUF,
            [pltpu.SemaphoreType.DMA] * NBUF,
        ],
        compiler_params=pltpu.CompilerParams(use_tc_tiling_on_sc=False),
    )
    def k(idx_hbm, table_hbm, out_hbm, idx_v, rows_v, sem_i, sem_g, sem_o):
        wid = lax.axis_index("s") * NC + lax.axis_index("c")
        base = wid * b_per_w

        def idx_start(c, b):
            pltpu.async_copy(
                idx_hbm.at[pl.ds(base + c * CHUNK, CHUNK)], idx_v.at[b], sem_i[b]
            )

        def idx_wait(b):
            pltpu.make_async_copy(
                idx_hbm.at[pl.ds(base, CHUNK)], idx_v.at[b], sem_i[b]
            ).wait()

        def gather_start(b):
            pltpu.async_copy(table_hbm.at[idx_v.at[b]], rows_v.at[b], sem_g[b])

        def gather_wait(b):
            pltpu.make_async_copy(
                table_hbm.at[idx_v.at[b]], rows_v.at[b], sem_g[b]
            ).wait()

        def out_start(c, b):
            pltpu.async_copy(
                rows_v.at[b], out_hbm.at[pl.ds(base + c * CHUNK, CHUNK)], sem_o[b]
            )

        def out_wait(b):
            pltpu.make_async_copy(
                rows_v.at[b], out_hbm.at[pl.ds(base, CHUNK)], sem_o[b]
            ).wait()

        idx_start(0, 0)
        idx_start(1, 1)
        idx_wait(0)
        gather_start(0)

        def group(g, carry):
            for b in range(NBUF):
                c = g * NBUF + b
                b1 = (b + 1) % NBUF
                b2 = (b + 2) % NBUF

                @pl.when(c >= NBUF - 1)
                def _():
                    out_wait(b1)

                @pl.when(c + 1 < n_chunks)
                def _():
                    idx_wait(b1)
                    gather_start(b1)

                @pl.when(c + 2 < n_chunks)
                def _():
                    idx_start(c + 2, b2)

                gather_wait(b)
                out_start(c, b)
            return carry

        lax.fori_loop(0, n_chunks // NBUF, group, 0)

        for j in range(1, NBUF):
            out_wait((n_chunks - j) % NBUF)

    return k(idx_flat, weight)


def kernel(x, weight):
    rows, cols = x.shape
    b_total = rows * cols
    flat = x.reshape(b_total).astype(jnp.int32)
    out = _gather_flat(flat, weight, b_total)
    return out.reshape(rows, cols, D)


# speedup vs baseline: 5.1800x; 1.0261x over previous
import functools

import jax
import jax.numpy as jnp
from jax import lax
from jax.experimental import pallas as pl
from jax.experimental.pallas import tpu as pltpu
from jax.experimental.pallas import tpu_sc as plsc

D = 32
NC = 2
NS = 16
NW = NC * NS
SUB = 512
TQ = 4


@jax.jit
def _gather_native(xp, weight):
    n_sub = xp.shape[0] // SUB
    per_w = n_sub // NW
    n_pairs = per_w // 2
    mesh = plsc.VectorSubcoreMesh(core_axis_name="c", subcore_axis_name="s")

    @functools.partial(
        pl.kernel,
        mesh=mesh,
        out_type=jax.ShapeDtypeStruct((200, 4, 128, 8, 128), jnp.float32),
        scratch_types=[
            pltpu.VMEM((2, SUB), jnp.int32),
            pltpu.VMEM((2, SUB, D), jnp.float32),
            pltpu.VMEM((2, TQ, 4, 8, 128), jnp.float32),
            [pltpu.SemaphoreType.DMA] * 2,
            [pltpu.SemaphoreType.DMA] * 2,
            [pltpu.SemaphoreType.DMA] * 2,
        ],
        compiler_params=pltpu.CompilerParams(
            use_tc_tiling_on_sc=False, needs_layout_passes=False
        ),
    )
    def k(xp_hbm, table_hbm, out_hbm, idx_v, rows_v, dst_v, sem_i, sem_g, sem_o):
        wid = lax.axis_index("s") * NC + lax.axis_index("c")
        m0 = wid * per_w
        iota16 = lax.iota(jnp.int32, 16)

        def idx_start(n, b):
            pltpu.async_copy(
                xp_hbm.at[pl.ds((m0 + n) * SUB, SUB)], idx_v.at[b], sem_i[b]
            )

        def idx_wait(b):
            pltpu.make_async_copy(
                xp_hbm.at[pl.ds(0, SUB)], idx_v.at[b], sem_i[b]
            ).wait()

        def gather_start(b):
            pltpu.async_copy(table_hbm.at[idx_v.at[b]], rows_v.at[b], sem_g[b])

        def gather_wait(b):
            pltpu.make_async_copy(
                table_hbm.at[idx_v.at[b]], rows_v.at[b], sem_g[b]
            ).wait()

        def transpose(b):
            rows = rows_v.at[b]
            for tq in range(TQ):
                base_row = tq * 128

                def dbody(i, carry):
                    vs = []
                    for u in range(2):
                        dd = 2 * i + u
                        cidx = jnp.full((16,), dd, jnp.int32)
                        for g in range(8):
                            ridx = iota16 + (base_row + g * 16)
                            vs.append(plsc.load_gather(rows, [ridx, cidx]))
                    for u in range(2):
                        dd = 2 * i + u
                        dtr = dd // 8
                        ddr = dd % 8
                        for g in range(8):
                            dst_v[b, tq, dtr, ddr, pl.ds(g * 16, 16)] = vs[
                                8 * u + g
                            ]
                    return carry

                lax.fori_loop(0, D // 2, dbody, 0)

        def out_start(n, b):
            m = m0 + n
            beta = m // 2
            t_base = (beta // 128) * 8 + (m % 2) * TQ
            btc = beta % 128
            for tq in range(TQ):
                for dtr in range(4):
                    pltpu.async_copy(
                        dst_v.at[b, tq, dtr],
                        out_hbm.at[t_base + tq, dtr, btc],
                        sem_o[b],
                    )

        def out_wait(b):
            for _ in range(TQ * 4):
                pltpu.make_async_copy(
                    dst_v.at[b, 0, 0], out_hbm.at[0, 0, 0], sem_o[b]
                ).wait()

        idx_start(0, 0)
        idx_start(1, 1)
        idx_wait(0)
        gather_start(0)

        def pair(p, carry):
            for nb in range(2):
                n = 2 * p + nb
                other = 1 - nb

                gather_wait(nb)

                @pl.when(n + 2 < per_w)
                def _():
                    idx_start(n + 2, nb)

                @pl.when(n + 1 < per_w)
                def _():
                    idx_wait(other)
                    gather_start(other)

                @pl.when(n >= 2)
                def _():
                    out_wait(nb)

                transpose(nb)
                out_start(n, nb)
            return carry

        lax.fori_loop(0, n_pairs, pair, 0)

        out_wait(0)
        out_wait(1)

    return k(xp, weight)


def kernel(x, weight):
    rows, cols = x.shape
    xp = (
        x.astype(jnp.int32)
        .reshape(128, 128, cols // 8, 8)
        .transpose(2, 0, 3, 1)
        .reshape(rows * cols)
    )
    out5 = _gather_native(xp, weight)
    return out5.transpose(2, 4, 0, 1, 3).reshape(rows, cols, D)


# speedup vs baseline: 5.2043x; 1.0047x over previous
import functools

import jax
import jax.numpy as jnp
from jax import lax
from jax.experimental import pallas as pl
from jax.experimental.pallas import tpu as pltpu
from jax.experimental.pallas import tpu_sc as plsc

D = 32
NC = 2
NS = 16
NW = NC * NS
SUB = 512
TQ = 4


@jax.jit
def _gather_native(xp, weight):
    n_sub = xp.shape[0] // SUB
    per_w = n_sub // NW
    n_pairs = per_w // 2
    mesh = plsc.VectorSubcoreMesh(core_axis_name="c", subcore_axis_name="s")

    @functools.partial(
        pl.kernel,
        mesh=mesh,
        out_type=jax.ShapeDtypeStruct((200, 4, 128, 8, 128), jnp.float32),
        scratch_types=[
            pltpu.VMEM((2, SUB), jnp.int32),
            pltpu.VMEM((2, SUB, D), jnp.float32),
            pltpu.VMEM((2, TQ, 4, 8, 128), jnp.float32),
            [pltpu.SemaphoreType.DMA] * 2,
            [pltpu.SemaphoreType.DMA] * 2,
            [pltpu.SemaphoreType.DMA] * 2,
        ],
        compiler_params=pltpu.CompilerParams(
            use_tc_tiling_on_sc=False, needs_layout_passes=False
        ),
    )
    def k(xp_hbm, table_hbm, out_hbm, idx_v, rows_v, dst_v, sem_i, sem_g, sem_o):
        wid = lax.axis_index("s") * NC + lax.axis_index("c")
        m0 = wid * per_w
        iota16 = lax.iota(jnp.int32, 16)

        def idx_start(n, b):
            pltpu.async_copy(
                xp_hbm.at[pl.ds((m0 + n) * SUB, SUB)], idx_v.at[b], sem_i[b]
            )

        def idx_wait(b):
            pltpu.make_async_copy(
                xp_hbm.at[pl.ds(0, SUB)], idx_v.at[b], sem_i[b]
            ).wait()

        def gather_start(b):
            pltpu.async_copy(table_hbm.at[idx_v.at[b]], rows_v.at[b], sem_g[b])

        def gather_wait(b):
            pltpu.make_async_copy(
                table_hbm.at[idx_v.at[b]], rows_v.at[b], sem_g[b]
            ).wait()

        def transpose(b):
            rows = rows_v.at[b]
            for tq in range(TQ):
                base_row = tq * 128

                def dbody(i, carry):
                    vs = []
                    for u in range(2):
                        dd = 2 * i + u
                        cidx = jnp.full((16,), dd, jnp.int32)
                        for g in range(8):
                            ridx = iota16 + (base_row + g * 16)
                            vs.append(plsc.load_gather(rows, [ridx, cidx]))
                    for u in range(2):
                        dd = 2 * i + u
                        dtr = dd // 8
                        ddr = dd % 8
                        for g in range(8):
                            dst_v[b, tq, dtr, ddr, pl.ds(g * 16, 16)] = vs[
                                8 * u + g
                            ]
                    return carry

                lax.fori_loop(0, D // 2, dbody, 0)

        def out_start(n, b):
            m = m0 + n
            beta = m // 2
            t_base = (beta // 128) * 8 + (m % 2) * TQ
            btc = beta % 128
            pltpu.async_copy(
                dst_v.at[b],
                out_hbm.at[pl.ds(t_base, TQ), :, btc],
                sem_o[b],
            )

        def out_wait(b):
            pltpu.make_async_copy(
                dst_v.at[b], out_hbm.at[pl.ds(0, TQ), :, 0], sem_o[b]
            ).wait()

        idx_start(0, 0)
        idx_start(1, 1)
        idx_wait(0)
        gather_start(0)

        def pair(p, carry):
            for nb in range(2):
                n = 2 * p + nb
                other = 1 - nb

                gather_wait(nb)

                @pl.when(n + 2 < per_w)
                def _():
                    idx_start(n + 2, nb)

                @pl.when(n + 1 < per_w)
                def _():
                    idx_wait(other)
                    gather_start(other)

                @pl.when(n >= 2)
                def _():
                    out_wait(nb)

                transpose(nb)
                out_start(n, nb)
            return carry

        lax.fori_loop(0, n_pairs, pair, 0)

        out_wait(0)
        out_wait(1)

    return k(xp, weight)


def kernel(x, weight):
    rows, cols = x.shape
    xp = (
        x.astype(jnp.int32)
        .reshape(128, 128, cols // 8, 8)
        .transpose(2, 0, 3, 1)
        .reshape(rows * cols)
    )
    out5 = _gather_native(xp, weight)
    return out5.transpose(2, 4, 0, 1, 3).reshape(rows, cols, D)


# speedup vs baseline: 14.6764x; 2.8200x over previous
import functools

import jax
import jax.numpy as jnp
from jax import lax
from jax.experimental import pallas as pl
from jax.experimental.pallas import tpu as pltpu
from jax.experimental.pallas import tpu_sc as plsc

D = 32
NC = 2
NS = 16
NW = NC * NS
SUB = 512
TQ = 4


@jax.jit
def _gather_native(xp, weight):
    n_sub = xp.shape[0] // SUB
    per_w = n_sub // NW
    n_pairs = per_w // 2
    mesh = plsc.VectorSubcoreMesh(core_axis_name="c", subcore_axis_name="s")

    @functools.partial(
        pl.kernel,
        mesh=mesh,
        out_type=jax.ShapeDtypeStruct((200, 4, 128, 8, 128), jnp.float32),
        scratch_types=[
            pltpu.VMEM((2, SUB), jnp.int32),
            pltpu.VMEM((2, SUB, D), jnp.float32),
            pltpu.VMEM((2, TQ, 4, 8, 128), jnp.float32),
            [pltpu.SemaphoreType.DMA] * 2,
            [pltpu.SemaphoreType.DMA] * 2,
            [pltpu.SemaphoreType.DMA] * 2,
        ],
        compiler_params=pltpu.CompilerParams(
            use_tc_tiling_on_sc=False, needs_layout_passes=False
        ),
    )
    def k(xp_hbm, table_hbm, out_hbm, idx_v, rows_v, dst_v, sem_i, sem_g, sem_o):
        wid = lax.axis_index("s") * NC + lax.axis_index("c")
        m0 = wid * per_w
        iota16 = lax.iota(jnp.int32, 16)

        def idx_start(n, b):
            pltpu.async_copy(
                xp_hbm.at[pl.ds((m0 + n) * SUB, SUB)], idx_v.at[b], sem_i[b]
            )

        def idx_wait(b):
            pltpu.make_async_copy(
                xp_hbm.at[pl.ds(0, SUB)], idx_v.at[b], sem_i[b]
            ).wait()

        def gather_start(b):
            pltpu.async_copy(table_hbm.at[idx_v.at[b]], rows_v.at[b], sem_g[b])

        def gather_wait(b):
            pltpu.make_async_copy(
                table_hbm.at[idx_v.at[b]], rows_v.at[b], sem_g[b]
            ).wait()

        def transpose(b):
            rows = rows_v.at[b]
            for tq in range(TQ):
                base_row = tq * 128

                def dbody(d0, carry):
                    cidx = (d0 + iota16) & (D - 1)
                    dtrv = cidx >> 3
                    ddrv = cidx & 7
                    vs = []
                    for g in range(8):
                        ridx = iota16 + (base_row + g * 16)
                        vs.append(plsc.load_gather(rows, [ridx, cidx]))
                    for g in range(8):
                        plsc.store_scatter(
                            dst_v.at[b, tq],
                            [dtrv, ddrv, iota16 + g * 16],
                            vs[g],
                        )
                    return carry

                lax.fori_loop(0, D, dbody, 0)

        def out_start(n, b):
            m = m0 + n
            beta = m // 2
            t_base = (beta // 128) * 8 + (m % 2) * TQ
            btc = beta % 128
            pltpu.async_copy(
                dst_v.at[b],
                out_hbm.at[pl.ds(t_base, TQ), :, btc],
                sem_o[b],
            )

        def out_wait(b):
            pltpu.make_async_copy(
                dst_v.at[b], out_hbm.at[pl.ds(0, TQ), :, 0], sem_o[b]
            ).wait()

        idx_start(0, 0)
        idx_start(1, 1)
        idx_wait(0)
        gather_start(0)

        def pair(p, carry):
            for nb in range(2):
                n = 2 * p + nb
                other = 1 - nb

                gather_wait(nb)

                @pl.when(n + 2 < per_w)
                def _():
                    idx_start(n + 2, nb)

                @pl.when(n + 1 < per_w)
                def _():
                    idx_wait(other)
                    gather_start(other)

                @pl.when(n >= 2)
                def _():
                    out_wait(nb)

                transpose(nb)
                out_start(n, nb)
            return carry

        lax.fori_loop(0, n_pairs, pair, 0)

        out_wait(0)
        out_wait(1)

    return k(xp, weight)


def kernel(x, weight):
    rows, cols = x.shape
    xp = (
        x.astype(jnp.int32)
        .reshape(128, 128, cols // 8, 8)
        .transpose(2, 0, 3, 1)
        .reshape(rows * cols)
    )
    out5 = _gather_native(xp, weight)
    return out5.transpose(2, 4, 0, 1, 3).reshape(rows, cols, D)
